# baseline (speedup 1.0000x reference)
import functools

import jax
import jax.numpy as jnp
from jax import lax
from jax.experimental import pallas as pl
from jax.experimental.pallas import tpu as pltpu
from jax.experimental.pallas import tpu_sc as plsc

_NUM_CORES = 2
_NUM_SUBCORES = 16
_NW = _NUM_CORES * _NUM_SUBCORES
_CHUNK = 128
_NCHUNKS = 20
_BLK = 128
_PITCH = 64
_D = 50
_SPG = 4
_VB = 32768


def _tc_pack(tt, n_out_rows):
    d, v = tt.shape
    grid = (n_out_rows * 2 + _VB - 1) // _VB
    half = _VB // 2

    def body(in_ref, out_ref):
        x = in_ref[...]
        eye = (lax.broadcasted_iota(jnp.int32, (d, d), 0)
               == lax.broadcasted_iota(jnp.int32, (d, d), 1)).astype(jnp.float32)
        xt = lax.dot_general(
            x, eye, (((0,), (0,)), ((), ())),
            preferred_element_type=jnp.float32,
        )
        out_ref[:, 0:d] = xt[:half]
        out_ref[:, _PITCH:_PITCH + d] = xt[half:]

    return pl.pallas_call(
        body,
        grid=(grid,),
        in_specs=[pl.BlockSpec((d, _VB), lambda i: (0, i))],
        out_specs=pl.BlockSpec((_VB // 2, 128), lambda i: (i, 0)),
        out_shape=jax.ShapeDtypeStruct((n_out_rows, 128), jnp.float32),
    )(tt)


def _sc_gather_extract(table64, p3d):
    n_out = _NW * _NCHUNKS * _CHUNK * _D
    mesh = plsc.VectorSubcoreMesh(core_axis_name="c", subcore_axis_name="s")
    stage_words = _SPG * _CHUNK * _D

    @functools.partial(
        pl.kernel,
        out_type=jax.ShapeDtypeStruct((n_out,), jnp.float32),
        mesh=mesh,
        scratch_types=[
            pltpu.VMEM((_NCHUNKS, _CHUNK), jnp.int32),
            pltpu.VMEM((_SPG, _CHUNK, _PITCH), jnp.float32),
            pltpu.VMEM((stage_words,), jnp.float32),
            pltpu.SemaphoreType.DMA,
            pltpu.SemaphoreType.DMA,
            pltpu.SemaphoreType.DMA,
            pltpu.SemaphoreType.DMA,
        ],
        compiler_params=pltpu.CompilerParams(
            use_tc_tiling_on_sc=False, needs_layout_passes=False
        ),
    )
    def gather_kernel(table_hbm, p_hbm, out_hbm, p_v, rows_v, stage_v,
                      sem0, sem1, sem2, sem3):
        wid = lax.axis_index("s") * _NUM_CORES + lax.axis_index("c")
        sems = (sem0, sem1, sem2, sem3)
        pltpu.sync_copy(p_hbm.at[wid], p_v)

        def group_body(g, carry):
            copies = [
                pltpu.async_copy(
                    table_hbm.at[p_v.at[_SPG * g + b]],
                    rows_v.at[b],
                    sems[b],
                )
                for b in range(_SPG)
            ]
            for b in range(_SPG):
                copies[b].wait()
                st_base = b * (_CHUNK * _D)
                for j in range(_CHUNK):
                    for t in (0, 16, 32, _D - 16):
                        stage_v[pl.ds(st_base + j * _D + t, 16)] = (
                            rows_v[b, j, pl.ds(t, 16)]
                        )
            off = wid * (_NCHUNKS * _CHUNK * _D) + g * stage_words
            pltpu.sync_copy(stage_v, out_hbm.at[pl.ds(off, stage_words)])
            return carry

        lax.fori_loop(0, _NCHUNKS // _SPG, group_body, None)

    return gather_kernel(table64, p3d)


def _mlp(win, W1, b1, W2):
    b, k = win.shape
    h = W1.shape[0]
    o = W2.shape[0]
    blk = 2048

    def body(win_ref, w1_ref, b1_ref, w2_ref, out_ref):
        z = lax.dot_general(
            win_ref[...], w1_ref[...],
            (((1,), (1,)), ((), ())),
            preferred_element_type=jnp.float32,
        )
        act = jax.nn.sigmoid(z + b1_ref[...])
        out_ref[...] = lax.dot_general(
            act, w2_ref[...],
            (((1,), (1,)), ((), ())),
            preferred_element_type=jnp.float32,
        )

    return pl.pallas_call(
        body,
        grid=(b // blk,),
        in_specs=[
            pl.BlockSpec((blk, k), lambda i: (i, 0)),
            pl.BlockSpec((h, k), lambda i: (0, 0)),
            pl.BlockSpec((1, h), lambda i: (0, 0)),
            pl.BlockSpec((o, h), lambda i: (0, 0)),
        ],
        out_specs=pl.BlockSpec((blk, o), lambda i: (i, 0)),
        out_shape=jax.ShapeDtypeStruct((b, o), jnp.float32),
    )(win, W1, b1.reshape(1, h), W2)


def kernel(x, table, W1, b1, W2):
    batch, win = x.shape
    v, d = table.shape
    xi = x.astype(jnp.int32).reshape(_NW, _NCHUNKS, _CHUNK)
    hb = _VB.bit_length() - 2
    p3d = (((xi >> (hb + 1)) << (hb + 1))
           + ((xi & ((1 << hb) - 1)) << 1)
           + ((xi >> hb) & 1))
    grid = (v + _VB - 1) // _VB
    table2 = _tc_pack(table.T, grid * (_VB // 2))
    table64 = table2.reshape(-1, _PITCH)
    packed = _sc_gather_extract(table64, p3d)
    win_emb = packed.reshape(batch, win * d)
    return _mlp(win_emb, W1, b1, W2)


# speedup vs baseline: 1.0017x; 1.0017x over previous
import functools

import jax
import jax.numpy as jnp
from jax import lax
from jax.experimental import pallas as pl
from jax.experimental.pallas import tpu as pltpu
from jax.experimental.pallas import tpu_sc as plsc

_NUM_CORES = 2
_NUM_SUBCORES = 16
_NW = _NUM_CORES * _NUM_SUBCORES
_CHUNK = 128
_NCHUNKS = 20
_PITCH = 64
_D = 50
_SPG = 4
_VB = 32768


def _tc_pack(tt, n_out_rows):
    d, v = tt.shape
    grid = (n_out_rows * 2 + _VB - 1) // _VB
    half = _VB // 2

    def body(in_ref, out_ref):
        x = in_ref[...]
        xt = x.T
        out_ref[:, 0:d] = xt[:half]
        out_ref[:, _PITCH:_PITCH + d] = xt[half:]

    return pl.pallas_call(
        body,
        grid=(grid,),
        in_specs=[pl.BlockSpec((d, _VB), lambda i: (0, i))],
        out_specs=pl.BlockSpec((_VB // 2, 128), lambda i: (i, 0)),
        out_shape=jax.ShapeDtypeStruct((n_out_rows, 128), jnp.float32),
    )(tt)


def _sc_gather_extract(table64, p3d):
    n_out = _NW * _NCHUNKS * _CHUNK * _D
    mesh = plsc.VectorSubcoreMesh(core_axis_name="c", subcore_axis_name="s")
    stage_words = _SPG * _CHUNK * _D

    @functools.partial(
        pl.kernel,
        out_type=jax.ShapeDtypeStruct((n_out,), jnp.float32),
        mesh=mesh,
        scratch_types=[
            pltpu.VMEM((_NCHUNKS, _CHUNK), jnp.int32),
            pltpu.VMEM((_SPG, _CHUNK, _PITCH), jnp.float32),
            pltpu.VMEM((stage_words,), jnp.float32),
            pltpu.SemaphoreType.DMA,
            pltpu.SemaphoreType.DMA,
            pltpu.SemaphoreType.DMA,
            pltpu.SemaphoreType.DMA,
        ],
        compiler_params=pltpu.CompilerParams(
            use_tc_tiling_on_sc=False, needs_layout_passes=False
        ),
    )
    def gather_kernel(table_hbm, p_hbm, out_hbm, p_v, rows_v, stage_v,
                      sem0, sem1, sem2, sem3):
        wid = lax.axis_index("s") * _NUM_CORES + lax.axis_index("c")
        sems = (sem0, sem1, sem2, sem3)
        pltpu.sync_copy(p_hbm.at[wid], p_v)

        def group_body(g, carry):
            copies = [
                pltpu.async_copy(
                    table_hbm.at[p_v.at[_SPG * g + b]],
                    rows_v.at[b],
                    sems[b],
                )
                for b in range(_SPG)
            ]
            for b in range(_SPG):
                copies[b].wait()
                st_base = b * (_CHUNK * _D)
                for j in range(_CHUNK):
                    for t in (0, 16, 32, _D - 16):
                        stage_v[pl.ds(st_base + j * _D + t, 16)] = (
                            rows_v[b, j, pl.ds(t, 16)]
                        )
            off = wid * (_NCHUNKS * _CHUNK * _D) + g * stage_words
            pltpu.sync_copy(stage_v, out_hbm.at[pl.ds(off, stage_words)])
            return carry

        lax.fori_loop(0, _NCHUNKS // _SPG, group_body, None)

    return gather_kernel(table64, p3d)


def _mlp(win, W1, b1, W2):
    b, k = win.shape
    h = W1.shape[0]
    o = W2.shape[0]
    blk = 2048

    def body(win_ref, w1_ref, b1_ref, w2_ref, out_ref):
        z = lax.dot_general(
            win_ref[...], w1_ref[...],
            (((1,), (1,)), ((), ())),
            preferred_element_type=jnp.float32,
        )
        act = jax.nn.sigmoid(z + b1_ref[...])
        out_ref[...] = lax.dot_general(
            act, w2_ref[...],
            (((1,), (1,)), ((), ())),
            preferred_element_type=jnp.float32,
        )

    return pl.pallas_call(
        body,
        grid=(b // blk,),
        in_specs=[
            pl.BlockSpec((blk, k), lambda i: (i, 0)),
            pl.BlockSpec((h, k), lambda i: (0, 0)),
            pl.BlockSpec((1, h), lambda i: (0, 0)),
            pl.BlockSpec((o, h), lambda i: (0, 0)),
        ],
        out_specs=pl.BlockSpec((blk, o), lambda i: (i, 0)),
        out_shape=jax.ShapeDtypeStruct((b, o), jnp.float32),
    )(win, W1, b1.reshape(1, h), W2)


def kernel(x, table, W1, b1, W2):
    batch, win = x.shape
    v, d = table.shape
    xi = x.astype(jnp.int32).reshape(_NW, _NCHUNKS, _CHUNK)
    hb = _VB.bit_length() - 2
    p3d = (((xi >> (hb + 1)) << (hb + 1))
           + ((xi & ((1 << hb) - 1)) << 1)
           + ((xi >> hb) & 1))
    grid = (v + _VB - 1) // _VB
    table2 = _tc_pack(table.T, grid * (_VB // 2))
    table64 = table2.reshape(-1, _PITCH)
    packed = _sc_gather_extract(table64, p3d)
    win_emb = packed.reshape(batch, win * d)
    return _mlp(win_emb, W1, b1, W2)
